# Initial kernel scaffold; baseline (speedup 1.0000x reference)
import functools

import jax
import jax.numpy as jnp
from jax import lax
from jax.experimental import pallas as pl
from jax.experimental.pallas import tpu as pltpu
from jax.experimental.pallas import tpu_sc as plsc

NT = 4096
C = 1024
DFF = 3584
E = 8
K = 2
NPAIR = NT * K
BLK = 256
NB = NPAIR // BLK + E
P = NB * BLK

NW = 32
GCH = 64
CCH = 32


def _router_body(x_ref, rw_ref, e1_ref, e2_ref, g1_ref, g2_ref, aux_ref):
    x = x_ref[...]
    rw = rw_ref[...]
    logits = lax.dot_general(x, rw, (((1,), (1,)), ((), ())),
                             preferred_element_type=jnp.float32)
    probs = jax.nn.softmax(logits, axis=-1)
    ids = lax.broadcasted_iota(jnp.int32, (NT, E), 1)
    p1 = jnp.max(probs, axis=1, keepdims=True)
    i1 = jnp.min(jnp.where(probs == p1, ids, E), axis=1, keepdims=True)
    masked = jnp.where(ids == i1, -1.0, probs)
    p2 = jnp.max(masked, axis=1, keepdims=True)
    i2 = jnp.min(jnp.where(masked == p2, ids, E), axis=1, keepdims=True)
    denom = p1 + p2
    e1_ref[...] = i1
    e2_ref[...] = i2
    g1_ref[...] = p1 / denom
    g2_ref[...] = p2 / denom
    sel = jnp.logical_or(ids == i1, ids == i2).astype(jnp.float32)
    f = jnp.sum(sel, axis=0) / (NT * K)
    p_mean = jnp.sum(probs, axis=0) / NT
    aux_ref[0, 0] = E * jnp.sum(f * p_mean)


def _router(x_flat, router_W):
    return pl.pallas_call(
        _router_body,
        out_shape=[
            jax.ShapeDtypeStruct((NT, 1), jnp.int32),
            jax.ShapeDtypeStruct((NT, 1), jnp.int32),
            jax.ShapeDtypeStruct((NT, 1), jnp.float32),
            jax.ShapeDtypeStruct((NT, 1), jnp.float32),
            jax.ShapeDtypeStruct((1, 1), jnp.float32),
        ],
    )(x_flat, router_W)


def _metadata(e1, e2, g1, g2):
    e = jnp.concatenate([e1, e2], axis=1).reshape(-1)
    g = jnp.concatenate([g1, g2], axis=1).reshape(-1)
    oh = (e[:, None] == jnp.arange(E)[None, :]).astype(jnp.int32)
    counts = jnp.sum(oh, axis=0)
    rank = jnp.sum((jnp.cumsum(oh, axis=0) - oh) * oh, axis=1)
    nblk = (counts + BLK - 1) // BLK
    used = jnp.cumsum(nblk)
    blk_start = jnp.concatenate([jnp.zeros((1,), used.dtype), used[:-1]])
    dest = jnp.take(blk_start, e) * BLK + rank
    src = jnp.zeros((P,), jnp.int32).at[dest].set(
        jnp.arange(NPAIR, dtype=jnp.int32) // K)
    gs = jnp.zeros((P,), jnp.float32).at[dest].set(g).reshape(P, 1)
    posk = dest.reshape(NT, K).astype(jnp.int32)
    pos0 = posk[:, 0]
    pos1 = posk[:, 1]
    block_expert = jnp.minimum(
        jnp.searchsorted(used, jnp.arange(NB), side="right"), E - 1
    ).astype(jnp.int32)
    return src, gs, pos0, pos1, block_expert


def _sc_gather(x_flat, src):
    mesh = plsc.VectorSubcoreMesh(core_axis_name="c", subcore_axis_name="s")
    rows_per_w = P // NW
    nch = rows_per_w // GCH

    @functools.partial(
        pl.kernel,
        mesh=mesh,
        out_type=jax.ShapeDtypeStruct((P, C), jnp.float32),
        scratch_types=[
            pltpu.VMEM((GCH,), jnp.int32),
            pltpu.VMEM((GCH, C), jnp.float32),
            pltpu.SemaphoreType.DMA,
        ],
    )
    def k(x_hbm, src_hbm, out_hbm, idx_v, rows_v, sem):
        wid = lax.axis_index("s") * 2 + lax.axis_index("c")
        base = wid * rows_per_w
        for ci in range(nch):
            b = base + ci * GCH
            pltpu.sync_copy(src_hbm.at[pl.ds(b, GCH)], idx_v)
            pltpu.async_copy(x_hbm.at[idx_v], rows_v, sem).wait()
            pltpu.sync_copy(rows_v, out_hbm.at[pl.ds(b, GCH)])

    return k(x_flat, src)


def _ffn_body(be_ref, xs_ref, w1_ref, w2_ref, w3_ref, gs_ref, out_ref):
    x = xs_ref[...].astype(jnp.bfloat16)
    w1 = w1_ref[0]
    w2 = w2_ref[0]
    w3 = w3_ref[0]
    a = lax.dot_general(x, w2, (((1,), (1,)), ((), ())),
                        preferred_element_type=jnp.float32)
    b = lax.dot_general(x, w1, (((1,), (1,)), ((), ())),
                        preferred_element_type=jnp.float32)
    h = (a * jax.nn.sigmoid(a) * b).astype(jnp.bfloat16)
    y = lax.dot_general(h, w3, (((1,), (1,)), ((), ())),
                        preferred_element_type=jnp.float32)
    out_ref[...] = y * gs_ref[...]


def _ffn(xs, w1b, w2b, w3b, gs, block_expert):
    grid_spec = pltpu.PrefetchScalarGridSpec(
        num_scalar_prefetch=1,
        grid=(NB,),
        in_specs=[
            pl.BlockSpec((BLK, C), lambda b, be: (b, 0)),
            pl.BlockSpec((1, DFF, C), lambda b, be: (be[b], 0, 0)),
            pl.BlockSpec((1, DFF, C), lambda b, be: (be[b], 0, 0)),
            pl.BlockSpec((1, C, DFF), lambda b, be: (be[b], 0, 0)),
            pl.BlockSpec((BLK, 1), lambda b, be: (b, 0)),
        ],
        out_specs=pl.BlockSpec((BLK, C), lambda b, be: (b, 0)),
    )
    return pl.pallas_call(
        _ffn_body,
        grid_spec=grid_spec,
        out_shape=jax.ShapeDtypeStruct((P, C), jnp.float32),
    )(block_expert, xs, w1b, w2b, w3b, gs)


def _sc_combine(ys, pos0, pos1):
    mesh = plsc.VectorSubcoreMesh(core_axis_name="c", subcore_axis_name="s")
    tok_per_w = NT // NW
    nch = tok_per_w // CCH

    @functools.partial(
        pl.kernel,
        mesh=mesh,
        out_type=jax.ShapeDtypeStruct((NT, C), jnp.float32),
        scratch_types=[
            pltpu.VMEM((CCH,), jnp.int32),
            pltpu.VMEM((CCH,), jnp.int32),
            pltpu.VMEM((CCH, C), jnp.float32),
            pltpu.VMEM((CCH, C), jnp.float32),
            pltpu.SemaphoreType.DMA,
        ],
    )
    def k(ys_hbm, p0_hbm, p1_hbm, out_hbm, i0_v, i1_v, a_v, b_v, sem):
        wid = lax.axis_index("s") * 2 + lax.axis_index("c")
        base = wid * tok_per_w
        for ci in range(nch):
            t0 = base + ci * CCH
            pltpu.sync_copy(p0_hbm.at[pl.ds(t0, CCH)], i0_v)
            pltpu.sync_copy(p1_hbm.at[pl.ds(t0, CCH)], i1_v)
            pltpu.async_copy(ys_hbm.at[i0_v], a_v, sem).wait()
            pltpu.async_copy(ys_hbm.at[i1_v], b_v, sem).wait()
            for r in range(CCH):
                def col_body(j, _, r=r):
                    cs = pl.multiple_of(j * 16, 16)
                    a_v[r, pl.ds(cs, 16)] = (a_v[r, pl.ds(cs, 16)]
                                             + b_v[r, pl.ds(cs, 16)])
                    return 0
                lax.fori_loop(0, C // 16, col_body, 0)
            pltpu.sync_copy(a_v, out_hbm.at[pl.ds(t0, CCH)])

    return k(ys, pos0, pos1)


def kernel(x, router_W, w1, w2, w3):
    Bb, Tt, Cc = x.shape
    x_flat = x.reshape(-1, Cc)
    e1, e2, g1, g2, aux = _router(x_flat, router_W)
    src, gs, pos0, pos1, block_expert = _metadata(e1, e2, g1, g2)
    w1b = w1.astype(jnp.bfloat16)
    w2b = w2.astype(jnp.bfloat16)
    w3b = w3.astype(jnp.bfloat16)
    xs = _sc_gather(x_flat, src)
    ys = _ffn(xs, w1b, w2b, w3b, gs, block_expert)
    y_flat = _sc_combine(ys, pos0, pos1)
    return (y_flat.reshape(Bb, Tt, Cc), aux[0, 0])


# speedup vs baseline: 1.9134x; 1.9134x over previous
import functools

import jax
import jax.numpy as jnp
from jax import lax
from jax.experimental import pallas as pl
from jax.experimental.pallas import tpu as pltpu
from jax.experimental.pallas import tpu_sc as plsc

NT = 4096
C = 1024
DFF = 3584
E = 8
K = 2
NPAIR = NT * K
BLK = 256
NB = NPAIR // BLK + E
P = NB * BLK

NW = 32
GCH = 64
CCH = 32


def _router_body(x_ref, rw_ref, e1_ref, e2_ref, g1_ref, g2_ref, aux_ref):
    x = x_ref[...]
    rw = rw_ref[...]
    logits = lax.dot_general(x, rw, (((1,), (1,)), ((), ())),
                             preferred_element_type=jnp.float32)
    probs = jax.nn.softmax(logits, axis=-1)
    ids = lax.broadcasted_iota(jnp.int32, (NT, E), 1)
    p1 = jnp.max(probs, axis=1, keepdims=True)
    i1 = jnp.min(jnp.where(probs == p1, ids, E), axis=1, keepdims=True)
    masked = jnp.where(ids == i1, -1.0, probs)
    p2 = jnp.max(masked, axis=1, keepdims=True)
    i2 = jnp.min(jnp.where(masked == p2, ids, E), axis=1, keepdims=True)
    denom = p1 + p2
    e1_ref[...] = i1
    e2_ref[...] = i2
    g1_ref[...] = p1 / denom
    g2_ref[...] = p2 / denom
    sel = jnp.logical_or(ids == i1, ids == i2).astype(jnp.float32)
    f = jnp.sum(sel, axis=0) / (NT * K)
    p_mean = jnp.sum(probs, axis=0) / NT
    aux_ref[...] = (E * jnp.sum(f * p_mean))[None, None]


def _router(x_flat, router_W):
    return pl.pallas_call(
        _router_body,
        out_shape=[
            jax.ShapeDtypeStruct((NT, 1), jnp.int32),
            jax.ShapeDtypeStruct((NT, 1), jnp.int32),
            jax.ShapeDtypeStruct((NT, 1), jnp.float32),
            jax.ShapeDtypeStruct((NT, 1), jnp.float32),
            jax.ShapeDtypeStruct((1, 1), jnp.float32),
        ],
    )(x_flat, router_W)


def _metadata(e1, e2, g1, g2):
    e = jnp.concatenate([e1, e2], axis=1).reshape(-1)
    g = jnp.concatenate([g1, g2], axis=1).reshape(-1)
    oh = (e[:, None] == jnp.arange(E)[None, :]).astype(jnp.int32)
    counts = jnp.sum(oh, axis=0)
    rank = jnp.sum((jnp.cumsum(oh, axis=0) - oh) * oh, axis=1)
    nblk = (counts + BLK - 1) // BLK
    used = jnp.cumsum(nblk)
    blk_start = jnp.concatenate([jnp.zeros((1,), used.dtype), used[:-1]])
    dest = jnp.take(blk_start, e) * BLK + rank
    src = jnp.zeros((P,), jnp.int32).at[dest].set(
        jnp.arange(NPAIR, dtype=jnp.int32) // K)
    gs = jnp.zeros((P,), jnp.float32).at[dest].set(g).reshape(P, 1)
    posk = dest.reshape(NT, K).astype(jnp.int32)
    pos0 = posk[:, 0]
    pos1 = posk[:, 1]
    block_expert = jnp.minimum(
        jnp.searchsorted(used, jnp.arange(NB), side="right"), E - 1
    ).astype(jnp.int32)
    return src, gs, pos0, pos1, block_expert


def _sc_gather(x_flat, src):
    mesh = plsc.VectorSubcoreMesh(core_axis_name="c", subcore_axis_name="s")
    rows_per_w = P // NW
    nch = rows_per_w // GCH

    @functools.partial(
        pl.kernel,
        mesh=mesh,
        out_type=jax.ShapeDtypeStruct((P, C), jnp.float32),
        scratch_types=[
            pltpu.VMEM((GCH,), jnp.int32),
            pltpu.VMEM((GCH, C), jnp.float32),
            pltpu.SemaphoreType.DMA,
        ],
    )
    def k(x_hbm, src_hbm, out_hbm, idx_v, rows_v, sem):
        wid = lax.axis_index("s") * 2 + lax.axis_index("c")
        base = wid * rows_per_w
        for ci in range(nch):
            b = base + ci * GCH
            pltpu.sync_copy(src_hbm.at[pl.ds(b, GCH)], idx_v)
            pltpu.async_copy(x_hbm.at[idx_v], rows_v, sem).wait()
            pltpu.sync_copy(rows_v, out_hbm.at[pl.ds(b, GCH)])

    return k(x_flat, src)


def _ffn_body(be_ref, xs_ref, w1_ref, w2_ref, w3_ref, gs_ref, out_ref):
    x = xs_ref[...].astype(jnp.bfloat16)
    w1 = w1_ref[0]
    w2 = w2_ref[0]
    w3 = w3_ref[0]
    a = lax.dot_general(x, w2, (((1,), (1,)), ((), ())),
                        preferred_element_type=jnp.float32)
    b = lax.dot_general(x, w1, (((1,), (1,)), ((), ())),
                        preferred_element_type=jnp.float32)
    h = (a * jax.nn.sigmoid(a) * b).astype(jnp.bfloat16)
    y = lax.dot_general(h, w3, (((1,), (1,)), ((), ())),
                        preferred_element_type=jnp.float32)
    out_ref[...] = y * gs_ref[...]


def _ffn(xs, w1b, w2b, w3b, gs, block_expert):
    grid_spec = pltpu.PrefetchScalarGridSpec(
        num_scalar_prefetch=1,
        grid=(NB,),
        in_specs=[
            pl.BlockSpec((BLK, C), lambda b, be: (b, 0)),
            pl.BlockSpec((1, DFF, C), lambda b, be: (be[b], 0, 0)),
            pl.BlockSpec((1, DFF, C), lambda b, be: (be[b], 0, 0)),
            pl.BlockSpec((1, C, DFF), lambda b, be: (be[b], 0, 0)),
            pl.BlockSpec((BLK, 1), lambda b, be: (b, 0)),
        ],
        out_specs=pl.BlockSpec((BLK, C), lambda b, be: (b, 0)),
    )
    return pl.pallas_call(
        _ffn_body,
        grid_spec=grid_spec,
        out_shape=jax.ShapeDtypeStruct((P, C), jnp.float32),
    )(block_expert, xs, w1b, w2b, w3b, gs)


def _sc_combine(ys, pos0, pos1):
    mesh = plsc.VectorSubcoreMesh(core_axis_name="c", subcore_axis_name="s")
    tok_per_w = NT // NW
    nch = tok_per_w // CCH

    @functools.partial(
        pl.kernel,
        mesh=mesh,
        out_type=jax.ShapeDtypeStruct((NT, C), jnp.float32),
        scratch_types=[
            pltpu.VMEM((CCH,), jnp.int32),
            pltpu.VMEM((CCH,), jnp.int32),
            pltpu.VMEM((CCH, C), jnp.float32),
            pltpu.VMEM((CCH, C), jnp.float32),
            pltpu.SemaphoreType.DMA,
        ],
    )
    def k(ys_hbm, p0_hbm, p1_hbm, out_hbm, i0_v, i1_v, a_v, b_v, sem):
        wid = lax.axis_index("s") * 2 + lax.axis_index("c")
        base = wid * tok_per_w
        for ci in range(nch):
            t0 = base + ci * CCH
            pltpu.sync_copy(p0_hbm.at[pl.ds(t0, CCH)], i0_v)
            pltpu.sync_copy(p1_hbm.at[pl.ds(t0, CCH)], i1_v)
            pltpu.async_copy(ys_hbm.at[i0_v], a_v, sem).wait()
            pltpu.async_copy(ys_hbm.at[i1_v], b_v, sem).wait()
            for r in range(CCH):
                def col_body(j, _, r=r):
                    cs = pl.multiple_of(j * 16, 16)
                    a_v[r, pl.ds(cs, 16)] = (a_v[r, pl.ds(cs, 16)]
                                             + b_v[r, pl.ds(cs, 16)])
                    return 0
                lax.fori_loop(0, C // 16, col_body, 0)
            pltpu.sync_copy(a_v, out_hbm.at[pl.ds(t0, CCH)])

    return k(ys, pos0, pos1)


def kernel(x, router_W, w1, w2, w3):
    Bb, Tt, Cc = x.shape
    x_flat = x.reshape(-1, Cc)
    e1, e2, g1, g2, aux = _router(x_flat, router_W)
    src, gs, pos0, pos1, block_expert = _metadata(e1, e2, g1, g2)
    w1b = w1.astype(jnp.bfloat16)
    w2b = w2.astype(jnp.bfloat16)
    w3b = w3.astype(jnp.bfloat16)
    xs = _sc_gather(x_flat, src)
    ys = _ffn(xs, w1b, w2b, w3b, gs, block_expert)
    y_flat = _sc_combine(ys, pos0, pos1)
    return (y_flat.reshape(Bb, Tt, Cc), aux[0, 0])
